# Initial kernel scaffold; baseline (speedup 1.0000x reference)
import jax
import jax.numpy as jnp
from jax.experimental import pallas as pl


def kernel(uids, iids, pos, neg, adj_indices, adj_values, params):
    raise NotImplementedError("write your pallas kernel here")


# speedup vs baseline: 9.4398x; 9.4398x over previous
import functools

import jax
import jax.numpy as jnp
from jax import lax
from jax.experimental import pallas as pl
from jax.experimental.pallas import tpu as pltpu
from jax.experimental.pallas import tpu_sc as plsc

N_U = 5000
N_I = 5000
D = 128
E = 320000
B = 4096
TEMP = 0.2
LAMBDA_1 = 0.2
LAMBDA_2 = 1e-07

NP = 5120
D2 = 256
BA = 512
NBLK = NP // BA

_CH = 320
_NCHUNK = NP // _CH
_CHW = _CH * NP
_DUMP = _CHW
_EPT = E // 16
_ESUB = 2000
_NSUB = _EPT // _ESUB
_ZB = 6400
_STRIPE = _CHW // 16


def _densify(rows, cols, vals):
    mesh = plsc.VectorSubcoreMesh(core_axis_name="c", subcore_axis_name="s")

    @functools.partial(
        pl.kernel,
        out_type=jax.ShapeDtypeStruct((NP * NP,), jnp.float32),
        mesh=mesh,
        scratch_types=[
            pltpu.VMEM((_ESUB,), jnp.int32),
            pltpu.VMEM((_ESUB,), jnp.int32),
            pltpu.VMEM((_ESUB,), jnp.float32),
            pltpu.VMEM((_ESUB,), jnp.int32),
            pltpu.VMEM((_ZB,), jnp.float32),
            pltpu.VMEM_SHARED((_CHW + 1024,), jnp.float32),
        ],
    )
    def k(rows_hbm, cols_hbm, vals_hbm, a_hbm, r_v, c_v, v_v, i_v, z_v, s_sh):
        cid = lax.axis_index("c")
        sid = lax.axis_index("s")

        def zfill(t, _):
            z_v[pl.ds(t * 16, 16)] = jnp.zeros((16,), jnp.float32)
            return _
        lax.fori_loop(0, _ZB // 16, zfill, 0)

        for kk in range(_NCHUNK // 2):
            chunk = 2 * kk + cid
            lo = chunk * _CH
            hi = lo + _CH
            for t in range(_STRIPE // _ZB):
                pltpu.sync_copy(z_v, s_sh.at[pl.ds(sid * _STRIPE + t * _ZB, _ZB)])
            plsc.subcore_barrier()

            def sub(s, _):
                ebase = sid * _EPT + s * _ESUB
                pltpu.sync_copy(rows_hbm.at[pl.ds(ebase, _ESUB)], r_v)
                pltpu.sync_copy(cols_hbm.at[pl.ds(ebase, _ESUB)], c_v)
                pltpu.sync_copy(vals_hbm.at[pl.ds(ebase, _ESUB)], v_v)

                def mkidx(t, __):
                    r = r_v[pl.ds(t * 16, 16)]
                    c = c_v[pl.ds(t * 16, 16)]
                    inr = (r >= lo) & (r < hi)
                    flat = jnp.where(inr, (r - lo) * NP + c,
                                     _DUMP + (c & 1023))
                    i_v[pl.ds(t * 16, 16)] = flat
                    return __
                lax.fori_loop(0, _ESUB // 16, mkidx, 0)
                pltpu.sync_copy(v_v, s_sh.at[i_v], add=True)
                return _
            lax.fori_loop(0, _NSUB, sub, 0)
            plsc.subcore_barrier()
            pltpu.sync_copy(
                s_sh.at[pl.ds(sid * _STRIPE, _STRIPE)],
                a_hbm.at[pl.ds(chunk * _CHW + sid * _STRIPE, _STRIPE)])
            plsc.subcore_barrier()

    return k(rows, cols, vals)


def _gather6(egn, edn, egs, eds, uids, iids, pos, neg):
    mesh = plsc.VectorSubcoreMesh(core_axis_name="c", subcore_axis_name="s")
    bpw = B // 32
    out_sd = jax.ShapeDtypeStruct((B, D), jnp.float32)

    @functools.partial(
        pl.kernel,
        out_type=(out_sd,) * 6,
        mesh=mesh,
        scratch_types=[
            pltpu.VMEM((bpw,), jnp.int32),
            pltpu.VMEM((bpw, D), jnp.float32),
            pltpu.SemaphoreType.DMA,
        ],
    )
    def k(egn_h, edn_h, egs_h, eds_h, u_h, i_h, p_h, n_h,
          o_egnu, o_ednp, o_ednn, o_egsu, o_edsi, o_edni,
          idx_v, rows_v, sem):
        wid = lax.axis_index("s") * 2 + lax.axis_index("c")
        base = wid * bpw
        for tab, ih, out in ((egn_h, u_h, o_egnu), (edn_h, p_h, o_ednp),
                             (edn_h, n_h, o_ednn), (egs_h, u_h, o_egsu),
                             (eds_h, i_h, o_edsi), (edn_h, i_h, o_edni)):
            pltpu.sync_copy(ih.at[pl.ds(base, bpw)], idx_v)
            pltpu.async_copy(tab.at[idx_v], rows_v, sem).wait()
            pltpu.sync_copy(rows_v, out.at[pl.ds(base, bpw)])

    return k(egn, edn, egs, eds, uids, iids, pos, neg)


def _prop_body(a_ref, u0_ref, v0_ref, usum_ref, vsum_ref,
               up, vp, uc, vc):
    l = pl.program_id(0)
    i = pl.program_id(1)
    j = pl.program_id(2)

    @pl.when((l == 0) & (i == 0) & (j == 0))
    def _init():
        up[...] = u0_ref[...]
        vp[...] = v0_ref[...]
        usum_ref[...] = u0_ref[...]
        vsum_ref[...] = v0_ref[...]
        uc[...] = jnp.zeros_like(uc)
        vc[...] = jnp.zeros_like(vc)

    @pl.when((l == 1) & (i == 0) & (j == 0))
    def _rotate():
        usum_ref[...] += uc[...]
        vsum_ref[...] += vc[...]
        up[...] = uc[...]
        vp[...] = vc[...]
        uc[...] = jnp.zeros_like(uc)
        vc[...] = jnp.zeros_like(vc)

    a = a_ref[...]
    vblk = vp[pl.ds(j * BA, BA), :]
    ublk = up[pl.ds(i * BA, BA), :]
    uc[pl.ds(i * BA, BA), :] += lax.dot_general(
        a, vblk, (((1,), (0,)), ((), ())), preferred_element_type=jnp.float32)
    vc[pl.ds(j * BA, BA), :] += lax.dot_general(
        a, ublk, (((0,), (0,)), ((), ())), preferred_element_type=jnp.float32)

    @pl.when((l == 1) & (i == NBLK - 1) & (j == NBLK - 1))
    def _fin():
        usum_ref[...] += uc[...]
        vsum_ref[...] += vc[...]


def _propagate(ad, u0p, v0p):
    sd = jax.ShapeDtypeStruct((NP, D2), jnp.float32)
    return pl.pallas_call(
        _prop_body,
        grid=(2, NBLK, NBLK),
        in_specs=[
            pl.BlockSpec((BA, BA), lambda l, i, j: (i, j)),
            pl.BlockSpec((NP, D2), lambda l, i, j: (0, 0)),
            pl.BlockSpec((NP, D2), lambda l, i, j: (0, 0)),
        ],
        out_specs=[
            pl.BlockSpec((NP, D2), lambda l, i, j: (0, 0)),
            pl.BlockSpec((NP, D2), lambda l, i, j: (0, 0)),
        ],
        out_shape=(sd, sd),
        scratch_shapes=[pltpu.VMEM((NP, D2), jnp.float32)] * 4,
        compiler_params=pltpu.CompilerParams(vmem_limit_bytes=100 * 2**20),
    )(ad, u0p, v0p)


def _dot_t(x, w):
    return lax.dot_general(x, w, (((1,), (1,)), ((), ())),
                           preferred_element_type=jnp.float32)


def _softplus(x):
    return jnp.maximum(x, 0.0) + jnp.log1p(jnp.exp(-jnp.abs(x)))


def _cons_body(egs_ref, egn_ref, wq1, bq1, wq2, bq2, wk1, bk1, wk2, bk2,
               out_ref):
    q = jnp.maximum(_dot_t(egs_ref[...], wq1[...]) + bq1[...], 0.0)
    q = _dot_t(q, wq2[...]) + bq2[...]
    k = jnp.maximum(_dot_t(egn_ref[...], wk1[...]) + bk1[...], 0.0)
    k = _dot_t(k, wk2[...]) + bk2[...]
    qn = jnp.sqrt(jnp.sum(q * q, axis=1, keepdims=True))
    kn = jnp.sqrt(jnp.sum(k * k, axis=1, keepdims=True))
    q = q / jnp.maximum(qn, 1e-12)
    k = k / jnp.maximum(kn, 1e-12)
    d = jnp.sum(q * k, axis=1)
    out_ref[0, 0] = jnp.sum((d - 1.0) ** 2) / N_U


def _consistency(egs, egn, qmlp, kmlp):
    args = [egs, egn,
            qmlp[0]["W"], qmlp[0]["b"][None, :], qmlp[1]["W"], qmlp[1]["b"][None, :],
            kmlp[0]["W"], kmlp[0]["b"][None, :], kmlp[1]["W"], kmlp[1]["b"][None, :]]
    return pl.pallas_call(
        _cons_body,
        out_specs=pl.BlockSpec(memory_space=pltpu.SMEM),
        out_shape=jax.ShapeDtypeStruct((1, 1), jnp.float32),
    )(*args)[0, 0]


def _bpr_body(u_ref, p_ref, n_ref, w1u, w1p, b1, w2, b2, w3, b3, out_ref):
    def score(e_ref):
        h = jnp.maximum(_dot_t(u_ref[...], w1u[...]) +
                        _dot_t(e_ref[...], w1p[...]) + b1[...], 0.0)
        h = jnp.maximum(_dot_t(h, w2[...]) + b2[...], 0.0)
        return jnp.sum(h * w3[...], axis=1) + b3[0, 0]
    sp = score(p_ref)
    sn = score(n_ref)
    lr = (jnp.sum(_softplus(-sp)) + jnp.sum(_softplus(sn)) +
          jnp.sum(_softplus(-(sp - sn)))) / B
    out_ref[0, 0] = lr


def _bpr(u_emb, pos_emb, neg_emb, cmlp):
    w1 = cmlp[0]["W"]
    args = [u_emb, pos_emb, neg_emb,
            w1[:, :D], w1[:, D:], cmlp[0]["b"][None, :],
            cmlp[1]["W"], cmlp[1]["b"][None, :],
            cmlp[2]["W"], cmlp[2]["b"][None, :]]
    return pl.pallas_call(
        _bpr_body,
        out_specs=pl.BlockSpec(memory_space=pltpu.SMEM),
        out_shape=jax.ShapeDtypeStruct((1, 1), jnp.float32),
    )(*args)[0, 0]


_CB = 512


def _contr_body(gsu_ref, egn_ref, dsi_ref, edn_ref, egnu_ref, edni_ref,
                out_ref):
    b = pl.program_id(0)

    @pl.when(b == 0)
    def _():
        out_ref[0, 0] = 0.0
        out_ref[0, 1] = 0.0

    lu = _dot_t(gsu_ref[...], egn_ref[...]) * (1.0 / TEMP)
    su = jnp.sum(jnp.exp(lu), axis=1)
    li = _dot_t(dsi_ref[...], edn_ref[...]) * (1.0 / TEMP)
    si = jnp.sum(jnp.exp(li), axis=1)
    neg = jnp.sum(jnp.log(su + 1e-8)) + jnp.sum(jnp.log(si + 1e-8))
    pu = jnp.clip(jnp.sum(gsu_ref[...] * egnu_ref[...], axis=1) * (1.0 / TEMP),
                  -5.0, 5.0)
    pi = jnp.clip(jnp.sum(dsi_ref[...] * edni_ref[...], axis=1) * (1.0 / TEMP),
                  -5.0, 5.0)
    out_ref[0, 0] += neg
    out_ref[0, 1] += jnp.sum(pu) + jnp.sum(pi)


def _contrastive(egs_u, egn, eds_i, edn, egn_u, edn_i):
    blk = lambda b: (b, 0)
    full = lambda b: (0, 0)
    out = pl.pallas_call(
        _contr_body,
        grid=(B // _CB,),
        in_specs=[
            pl.BlockSpec((_CB, D), blk),
            pl.BlockSpec((N_U, D), full),
            pl.BlockSpec((_CB, D), blk),
            pl.BlockSpec((N_I, D), full),
            pl.BlockSpec((_CB, D), blk),
            pl.BlockSpec((_CB, D), blk),
        ],
        out_specs=pl.BlockSpec(memory_space=pltpu.SMEM),
        out_shape=jax.ShapeDtypeStruct((1, 2), jnp.float32),
        compiler_params=pltpu.CompilerParams(vmem_limit_bytes=100 * 2**20),
    )(egs_u, egn, eds_i, edn, egn_u, edn_i)
    neg_sum, pos_sum = out[0, 0], out[0, 1]
    return -pos_sum / B + neg_sum / B


def _sumsq_body(x_ref, out_ref):
    c = pl.program_id(0)

    @pl.when(c == 0)
    def _():
        out_ref[0, 0] = 0.0

    x = x_ref[...]
    out_ref[0, 0] += jnp.sum(x * x)


def _sumsq(vec):
    n = vec.shape[0]
    rows = 2048
    cols = 1024
    chunk = rows * cols
    nchunk = -(-n // chunk)
    pad = nchunk * chunk - n
    x = jnp.pad(vec, (0, pad)).reshape(nchunk * rows, cols)
    return pl.pallas_call(
        _sumsq_body,
        grid=(nchunk,),
        in_specs=[pl.BlockSpec((rows, cols), lambda c: (c, 0))],
        out_specs=pl.BlockSpec(memory_space=pltpu.SMEM),
        out_shape=jax.ShapeDtypeStruct((1, 1), jnp.float32),
    )(x)[0, 0]


def kernel(uids, iids, pos, neg, adj_indices, adj_values, params):
    rows = adj_indices[0]
    cols = adj_indices[1]

    ad = _densify(rows, cols, adj_values).reshape(NP, NP)

    u0 = jnp.concatenate([params["E_g_GNN_0"], params["E_g_SVD_0"]], axis=1)
    v0 = jnp.concatenate([params["E_d_GNN_0"], params["E_d_SVD_0"]], axis=1)
    u0p = jnp.pad(u0, ((0, NP - N_U), (0, 0)))
    v0p = jnp.pad(v0, ((0, NP - N_I), (0, 0)))

    usum, vsum = _propagate(ad, u0p, v0p)
    egn = usum[:N_U, :D]
    egs = usum[:N_U, D:]
    edn = vsum[:N_I, :D]
    eds = vsum[:N_I, D:]

    egn_u, edn_p, edn_n, egs_u, eds_i, edn_i = _gather6(
        egn, edn, egs, eds, uids, iids, pos, neg)

    cons = _consistency(egs, egn, params["query_mlp"], params["key_mlp"])
    loss_r = _bpr(egn_u, edn_p, edn_n, params["concat_mlp"])
    loss_s = _contrastive(egs_u, egn, eds_i, edn, egn_u, edn_i)

    reg_vec = jnp.concatenate(
        [jnp.ravel(p) for p in jax.tree_util.tree_leaves(params)])
    loss_reg = _sumsq(reg_vec) * LAMBDA_2

    ls = LAMBDA_1 * loss_s
    loss = loss_reg + ls + cons + loss_r
    return (loss, loss_r, ls)
